# baseline (speedup 1.0000x reference)
import functools

import jax
import jax.numpy as jnp
from jax import lax
from jax.experimental import pallas as pl
from jax.experimental.pallas import tpu as pltpu
from jax.experimental.pallas import tpu_sc as plsc

DIM = 64
LANES = 16
NUM_CORES = 2
NUM_SUBCORES = 16
NUM_WORKERS = NUM_CORES * NUM_SUBCORES
CHUNK = 128
UNROLL = 8


def _sc_transe(total):
    per_w = total // NUM_WORKERS
    n_chunks = per_w // CHUNK
    groups = CHUNK // LANES
    assert n_chunks % 2 == 0

    mesh = plsc.VectorSubcoreMesh(
        core_axis_name="c", subcore_axis_name="s",
        num_cores=NUM_CORES, num_subcores=NUM_SUBCORES)

    @functools.partial(
        pl.kernel,
        out_type=jax.ShapeDtypeStruct((total,), jnp.float32),
        mesh=mesh,
        compiler_params=pltpu.CompilerParams(
            needs_layout_passes=False, use_tc_tiling_on_sc=False),
        scratch_types=[
            pltpu.VMEM((n_chunks, CHUNK), jnp.int32),
            pltpu.VMEM((n_chunks, CHUNK), jnp.int32),
            pltpu.VMEM((n_chunks, CHUNK), jnp.int32),
            pltpu.VMEM((CHUNK, DIM), jnp.float32),
            pltpu.VMEM((CHUNK, DIM), jnp.float32),
            pltpu.VMEM((CHUNK, DIM), jnp.float32),
            pltpu.VMEM((CHUNK, DIM), jnp.float32),
            pltpu.VMEM((CHUNK, DIM), jnp.float32),
            pltpu.VMEM((CHUNK, DIM), jnp.float32),
            pltpu.VMEM((per_w,), jnp.float32),
            pltpu.SemaphoreType.DMA,
            pltpu.SemaphoreType.DMA,
        ],
    )
    def k(hidx_hbm, ridx_hbm, tidx_hbm, ent_hbm, rel_hbm, out_hbm,
          hidx_v, ridx_v, tidx_v, h0, r0, t0, h1, r1, t1, out_v,
          sem0, sem1):
        wid = lax.axis_index("s") * NUM_CORES + lax.axis_index("c")
        row0 = wid * n_chunks
        pltpu.sync_copy(hidx_hbm.at[pl.ds(row0, n_chunks)], hidx_v)
        pltpu.sync_copy(ridx_hbm.at[pl.ds(row0, n_chunks)], ridx_v)
        pltpu.sync_copy(tidx_hbm.at[pl.ds(row0, n_chunks)], tidx_v)

        lane = jnp.arange(LANES, dtype=jnp.int32)
        bufs = ((h0, r0, t0, sem0), (h1, r1, t1, sem1))

        def issue(j, buf):
            h_b, r_b, t_b, sem = buf
            pltpu.async_copy(ent_hbm.at[hidx_v.at[j]], h_b, sem)
            pltpu.async_copy(rel_hbm.at[ridx_v.at[j]], r_b, sem)
            pltpu.async_copy(ent_hbm.at[tidx_v.at[j]], t_b, sem)

        def drain_compute(j, buf):
            h_b, r_b, t_b, sem = buf
            pltpu.make_async_copy(ent_hbm.at[hidx_v.at[0]], h_b, sem).wait()
            pltpu.make_async_copy(rel_hbm.at[ridx_v.at[0]], r_b, sem).wait()
            pltpu.make_async_copy(ent_hbm.at[tidx_v.at[0]], t_b, sem).wait()

            for g in range(groups):
                rows = g * LANES + lane

                def d_body(d0, acc, rows=rows):
                    for u in range(UNROLL):
                        col = jnp.full((LANES,), d0 * UNROLL + u,
                                       dtype=jnp.int32)
                        hv = plsc.load_gather(h_b, [rows, col])
                        rv = plsc.load_gather(r_b, [rows, col])
                        tv = plsc.load_gather(t_b, [rows, col])
                        acc = acc + jnp.abs(hv + rv - tv)
                    return acc

                acc = lax.fori_loop(
                    0, DIM // UNROLL, d_body,
                    jnp.zeros((LANES,), jnp.float32))
                out_v[pl.ds(j * CHUNK + g * LANES, LANES)] = acc

        issue(0, bufs[0])
        for j in range(n_chunks):
            if j + 1 < n_chunks:
                issue(j + 1, bufs[(j + 1) % 2])
            drain_compute(j, bufs[j % 2])

        pltpu.sync_copy(out_v, out_hbm.at[pl.ds(wid * per_w, per_w)])

    return k


def kernel(positive_triplets, negative_triplets, entities_emb, relations_emb):
    batch = positive_triplets.shape[0]
    total = 2 * batch
    trip = jnp.concatenate(
        [positive_triplets, negative_triplets], axis=0).astype(jnp.int32)
    n_rows = total // CHUNK
    hidx = trip[:, 0].reshape(n_rows, CHUNK)
    ridx = trip[:, 1].reshape(n_rows, CHUNK)
    tidx = trip[:, 2].reshape(n_rows, CHUNK)

    out = _sc_transe(total)(hidx, ridx, tidx, entities_emb, relations_emb)
    return out[:batch], out[batch:]
